# baseline (speedup 1.0000x reference)
import jax
import jax.numpy as jnp
from jax import lax
from jax.experimental import pallas as pl

_N = 5000
_CONF = 0.1
_IOU = 0.6
_B = 512
_NP = 5120
_K = _NP // _B


def _nms_kernel(g_ref, w_ref):
    tri = (
        lax.broadcasted_iota(jnp.int32, (_B, _B), 0)
        < lax.broadcasted_iota(jnp.int32, (_B, _B), 1)
    ).astype(jnp.float32)

    for k in range(_K):
        lo = k * _B
        hi = lo + _B
        rx1 = g_ref[0:hi, 0:1]
        ry1 = g_ref[0:hi, 1:2]
        rx2 = g_ref[0:hi, 2:3]
        ry2 = g_ref[0:hi, 3:4]
        cx1 = g_ref[lo:hi, 0:1].reshape(1, _B)
        cy1 = g_ref[lo:hi, 1:2].reshape(1, _B)
        cx2 = g_ref[lo:hi, 2:3].reshape(1, _B)
        cy2 = g_ref[lo:hi, 3:4].reshape(1, _B)
        sblk = g_ref[lo:hi, 4:5].reshape(1, _B)

        iw = jnp.maximum(jnp.minimum(rx2, cx2) - jnp.maximum(rx1, cx1), 0.0)
        ih = jnp.maximum(jnp.minimum(ry2, cy2) - jnp.maximum(ry1, cy1), 0.0)
        inter = iw * ih
        area_r = (rx2 - rx1) * (ry2 - ry1) + 1e-9
        area_c = (cx2 - cx1) * (cy2 - cy1)
        iou = inter / (area_r + area_c - inter)
        m = (iou > _IOU).astype(jnp.float32)

        vblk = (sblk > _CONF).astype(jnp.float32)
        if k > 0:
            keep_earlier = w_ref[0:1, 0:lo]
            cross = jnp.dot(
                (keep_earlier > 0.0).astype(jnp.float32),
                m[0:lo, :],
                preferred_element_type=jnp.float32,
            )
            vblk = vblk * (cross < 0.5).astype(jnp.float32)

        mkk = m[lo:hi, :] * tri

        def body(carry):
            kb, _ = carry
            sup = jnp.dot(kb, mkk, preferred_element_type=jnp.float32)
            kbn = vblk * (sup < 0.5).astype(jnp.float32)
            return kbn, jnp.any(kbn != kb)

        def cond(carry):
            return carry[1]

        kb, _ = lax.while_loop(cond, body, (vblk, jnp.bool_(True)))
        w_ref[0:1, lo:hi] = kb * sblk


def kernel(boxes, scores):
    cx = boxes[:, 0] * 640.0
    cy = boxes[:, 1] * 640.0
    w = boxes[:, 2] * 64.0 + 4.0
    h = boxes[:, 3] * 64.0 + 4.0
    x1 = cx - w / 2
    y1 = cy - h / 2
    x2 = cx + w / 2
    y2 = cy + h / 2

    valid = scores > _CONF
    order = jnp.argsort(jnp.where(valid, -scores, jnp.inf))

    zeros = jnp.zeros((_N,), jnp.float32)
    packed = jnp.stack([x1, y1, x2, y2, scores, zeros, zeros, zeros], axis=-1)
    g = packed[order]
    g_p = jnp.pad(g, ((0, _NP - _N), (0, 0)))

    wsorted = pl.pallas_call(
        _nms_kernel,
        out_shape=jax.ShapeDtypeStruct((1, _NP), jnp.float32),
    )(g_p)

    worig = zeros.at[order].set(wsorted[0, :_N])
    bxyxy = jnp.stack([x1, y1, x2, y2], axis=-1)
    return bxyxy * worig[:, None]


# speedup vs baseline: 1.4966x; 1.4966x over previous
import jax
import jax.numpy as jnp
from jax import lax
from jax.experimental import pallas as pl
from jax.experimental.pallas import tpu as pltpu

_N = 5000
_CONF = 0.1
_IOU = 0.6
_B = 512
_NP = 5120
_K = _NP // _B


def _nms_kernel(g_ref, w_ref, cols_ref):
    tri = (
        lax.broadcasted_iota(jnp.int32, (_B, _B), 0)
        < lax.broadcasted_iota(jnp.int32, (_B, _B), 1)
    ).astype(jnp.float32)

    for k in range(_K):
        lo = k * _B
        hi = lo + _B
        cx1 = g_ref[0:1, lo:hi]
        cy1 = g_ref[1:2, lo:hi]
        cx2 = g_ref[2:3, lo:hi]
        cy2 = g_ref[3:4, lo:hi]
        cols_ref[lo:hi, 0:1] = cx1.reshape(_B, 1)
        cols_ref[lo:hi, 1:2] = cy1.reshape(_B, 1)
        cols_ref[lo:hi, 2:3] = cx2.reshape(_B, 1)
        cols_ref[lo:hi, 3:4] = cy2.reshape(_B, 1)
        rx1 = cols_ref[0:hi, 0:1]
        ry1 = cols_ref[0:hi, 1:2]
        rx2 = cols_ref[0:hi, 2:3]
        ry2 = cols_ref[0:hi, 3:4]

        iw = jnp.maximum(jnp.minimum(rx2, cx2) - jnp.maximum(rx1, cx1), 0.0)
        ih = jnp.maximum(jnp.minimum(ry2, cy2) - jnp.maximum(ry1, cy1), 0.0)
        inter = iw * ih
        area_r = (rx2 - rx1) * (ry2 - ry1) + 1e-9
        area_c = (cx2 - cx1) * (cy2 - cy1)
        iou = inter / (area_r + area_c - inter)
        m = (iou > _IOU).astype(jnp.float32)

        sblk = jnp.maximum(-g_ref[4:5, lo:hi], 0.0)
        vblk = (sblk > _CONF).astype(jnp.float32)
        if k > 0:
            keep_earlier = w_ref[0:1, 0:lo]
            cross = jnp.dot(
                (keep_earlier > 0.0).astype(jnp.float32),
                m[0:lo, :],
                preferred_element_type=jnp.float32,
            )
            vblk = vblk * (cross < 0.5).astype(jnp.float32)

        mkk = m[lo:hi, :] * tri

        def body(carry):
            kb, _ = carry
            sup = jnp.dot(kb, mkk, preferred_element_type=jnp.float32)
            kbn = vblk * (sup < 0.5).astype(jnp.float32)
            return kbn, jnp.any(kbn != kb)

        def cond(carry):
            return carry[1]

        kb, _ = lax.while_loop(cond, body, (vblk, jnp.bool_(True)))
        w_ref[0:1, lo:hi] = kb * sblk


def kernel(boxes, scores):
    cx = boxes[:, 0] * 640.0
    cy = boxes[:, 1] * 640.0
    w = boxes[:, 2] * 64.0 + 4.0
    h = boxes[:, 3] * 64.0 + 4.0
    x1 = cx - w / 2
    y1 = cy - h / 2
    x2 = cx + w / 2
    y2 = cy + h / 2

    valid = scores > _CONF
    key = jnp.where(valid, -scores, jnp.inf)
    iota = lax.iota(jnp.int32, _N)
    keys, x1s, y1s, x2s, y2s, order = lax.sort(
        (key, x1, y1, x2, y2, iota), num_keys=1, is_stable=True
    )

    g = jnp.stack([x1s, y1s, x2s, y2s, keys])
    g_p = jnp.pad(g, ((0, 0), (0, _NP - _N)))

    wsorted = pl.pallas_call(
        _nms_kernel,
        out_shape=jax.ShapeDtypeStruct((1, _NP), jnp.float32),
        scratch_shapes=[pltpu.VMEM((_NP, 8), jnp.float32)],
    )(g_p)

    worig = jnp.zeros((_N,), jnp.float32).at[order].set(wsorted[0, :_N])
    bxyxy = jnp.stack([x1, y1, x2, y2], axis=-1)
    return bxyxy * worig[:, None]


# speedup vs baseline: 1.8845x; 1.2592x over previous
import jax
import jax.numpy as jnp
from jax import lax
from jax.experimental import pallas as pl
from jax.experimental.pallas import tpu as pltpu

_N = 5000
_CONF = 0.1
_IOU = 0.6
_B = 512
_NP = 5120
_K = _NP // _B


def _nms_kernel(g_ref, w_ref, cols_ref):
    tri = (
        lax.broadcasted_iota(jnp.int32, (_B, _B), 0)
        < lax.broadcasted_iota(jnp.int32, (_B, _B), 1)
    ).astype(jnp.float32)

    for k in range(_K):
        lo = k * _B
        hi = lo + _B
        cx1 = g_ref[0:1, lo:hi]
        cy1 = g_ref[1:2, lo:hi]
        cx2 = g_ref[2:3, lo:hi]
        cy2 = g_ref[3:4, lo:hi]
        cols_ref[lo:hi, 0:1] = cx1.reshape(_B, 1)
        cols_ref[lo:hi, 1:2] = cy1.reshape(_B, 1)
        cols_ref[lo:hi, 2:3] = cx2.reshape(_B, 1)
        cols_ref[lo:hi, 3:4] = cy2.reshape(_B, 1)
        rx1 = cols_ref[0:hi, 0:1]
        ry1 = cols_ref[0:hi, 1:2]
        rx2 = cols_ref[0:hi, 2:3]
        ry2 = cols_ref[0:hi, 3:4]

        iw = jnp.maximum(jnp.minimum(rx2, cx2) - jnp.maximum(rx1, cx1), 0.0)
        ih = jnp.maximum(jnp.minimum(ry2, cy2) - jnp.maximum(ry1, cy1), 0.0)
        inter = iw * ih
        area_r = (rx2 - rx1) * (ry2 - ry1) + 1e-9
        area_c = (cx2 - cx1) * (cy2 - cy1)
        iou = inter / (area_r + area_c - inter)
        m = (iou > _IOU).astype(jnp.float32)

        sblk = jnp.maximum(-g_ref[4:5, lo:hi], 0.0)
        vblk = (sblk > _CONF).astype(jnp.float32)
        if k > 0:
            keep_earlier = w_ref[0:1, 0:lo]
            cross = jnp.dot(
                (keep_earlier > 0.0).astype(jnp.float32),
                m[0:lo, :],
                preferred_element_type=jnp.float32,
            )
            vblk = vblk * (cross < 0.5).astype(jnp.float32)

        mkk = m[lo:hi, :] * tri

        def body(carry):
            kb, _ = carry
            sup = jnp.dot(kb, mkk, preferred_element_type=jnp.float32)
            kbn = vblk * (sup < 0.5).astype(jnp.float32)
            return kbn, jnp.any(kbn != kb)

        def cond(carry):
            return carry[1]

        kb, _ = lax.while_loop(cond, body, (vblk, jnp.bool_(True)))
        w_ref[0:1, lo:hi] = kb * sblk


def kernel(boxes, scores):
    cx = boxes[:, 0] * 640.0
    cy = boxes[:, 1] * 640.0
    w = boxes[:, 2] * 64.0 + 4.0
    h = boxes[:, 3] * 64.0 + 4.0
    x1 = cx - w / 2
    y1 = cy - h / 2
    x2 = cx + w / 2
    y2 = cy + h / 2

    valid = scores > _CONF
    key = jnp.where(valid, -scores, jnp.inf)
    iota = lax.iota(jnp.int32, _N)
    keys, x1s, y1s, x2s, y2s, order = lax.sort(
        (key, x1, y1, x2, y2, iota), num_keys=1, is_stable=True
    )

    g = jnp.stack([x1s, y1s, x2s, y2s, keys])
    g_p = jnp.pad(g, ((0, 0), (0, _NP - _N)))

    wsorted = pl.pallas_call(
        _nms_kernel,
        out_shape=jax.ShapeDtypeStruct((1, _NP), jnp.float32),
        scratch_shapes=[pltpu.VMEM((_NP, 8), jnp.float32)],
    )(g_p)

    _, worig = lax.sort((order, wsorted[0, :_N]), num_keys=1, is_stable=False)
    bxyxy = jnp.stack([x1, y1, x2, y2], axis=-1)
    return bxyxy * worig[:, None]


# speedup vs baseline: 2.1129x; 1.1212x over previous
import jax
import jax.numpy as jnp
from jax import lax
from jax.experimental import pallas as pl

_N = 5000
_CONF = 0.1
_IOU = 0.6
_B = 512
_NP = 5120
_K = _NP // _B


def _nms_kernel(g_ref, w_ref):
    tril = (
        lax.broadcasted_iota(jnp.int32, (_B, _B), 0)
        > lax.broadcasted_iota(jnp.int32, (_B, _B), 1)
    ).astype(jnp.float32)

    for k in range(_K):
        lo = k * _B
        hi = lo + _B
        bx1 = g_ref[0:1, lo:hi].reshape(_B, 1)
        by1 = g_ref[1:2, lo:hi].reshape(_B, 1)
        bx2 = g_ref[2:3, lo:hi].reshape(_B, 1)
        by2 = g_ref[3:4, lo:hi].reshape(_B, 1)
        ax1 = g_ref[0:1, 0:hi]
        ay1 = g_ref[1:2, 0:hi]
        ax2 = g_ref[2:3, 0:hi]
        ay2 = g_ref[3:4, 0:hi]

        iw = jnp.maximum(jnp.minimum(bx2, ax2) - jnp.maximum(bx1, ax1), 0.0)
        ih = jnp.maximum(jnp.minimum(by2, ay2) - jnp.maximum(by1, ay1), 0.0)
        inter = iw * ih
        area_b = (bx2 - bx1) * (by2 - by1) + 1e-9
        area_a = (ax2 - ax1) * (ay2 - ay1)
        iou = inter / (area_b + area_a - inter)
        m = (iou > _IOU).astype(jnp.float32)

        sblk = jnp.maximum(-g_ref[4:5, lo:hi], 0.0).reshape(_B, 1)
        vblk = (sblk > _CONF).astype(jnp.float32)
        if k > 0:
            keep_earlier = (w_ref[0:lo, 0:1] > 0.0).astype(jnp.float32)
            cross = jnp.dot(
                m[:, 0:lo], keep_earlier, preferred_element_type=jnp.float32
            )
            vblk = vblk * (cross < 0.5).astype(jnp.float32)

        mll = m[:, lo:hi] * tril

        def body(carry):
            kb, _ = carry
            sup = jnp.dot(mll, kb, preferred_element_type=jnp.float32)
            kbn = vblk * (sup < 0.5).astype(jnp.float32)
            return kbn, jnp.any(kbn != kb)

        def cond(carry):
            return carry[1]

        kb, _ = lax.while_loop(cond, body, (vblk, jnp.bool_(True)))
        w_ref[lo:hi, 0:1] = kb * sblk


def kernel(boxes, scores):
    cx = boxes[:, 0] * 640.0
    cy = boxes[:, 1] * 640.0
    w = boxes[:, 2] * 64.0 + 4.0
    h = boxes[:, 3] * 64.0 + 4.0
    x1 = cx - w / 2
    y1 = cy - h / 2
    x2 = cx + w / 2
    y2 = cy + h / 2

    valid = scores > _CONF
    key = jnp.where(valid, -scores, jnp.inf)
    iota = lax.iota(jnp.int32, _N)
    keys, x1s, y1s, x2s, y2s, order = lax.sort(
        (key, x1, y1, x2, y2, iota), num_keys=1, is_stable=True
    )

    g = jnp.stack([x1s, y1s, x2s, y2s, keys])
    g_p = jnp.pad(g, ((0, 0), (0, _NP - _N)))

    wsorted = pl.pallas_call(
        _nms_kernel,
        out_shape=jax.ShapeDtypeStruct((_NP, 1), jnp.float32),
    )(g_p)

    _, worig = lax.sort((order, wsorted[:_N, 0]), num_keys=1, is_stable=False)
    bxyxy = jnp.stack([x1, y1, x2, y2], axis=-1)
    return bxyxy * worig[:, None]
